# baseline (speedup 1.0000x reference)
import functools

import jax
import jax.numpy as jnp
from jax import lax
from jax.experimental import pallas as pl
from jax.experimental.pallas import tpu as pltpu
from jax.experimental.pallas import tpu_sc as plsc

_N = 10000
_E = 320000
_NC = 2
_NS = 16
_NW = _NC * _NS
_EPT = _E // _NW
_RPT = _N // _NS
_DEGK = 16


def _mesh():
    return plsc.VectorSubcoreMesh(
        core_axis_name="c", subcore_axis_name="s", num_cores=_NC, num_subcores=_NS
    )


def _fill2d(ref, rows, d, value):
    vec = jnp.full((16,), value, jnp.float32)
    nl = d // 16

    def body(i, carry):
        r = i // nl
        c = i - r * nl
        ref[r, pl.ds(c * 16, 16)] = vec
        return carry

    lax.fori_loop(0, rows * nl, body, 0)


def _zero_my_slice(acc_sh, zbuf, sid, d, chunk):
    _fill2d(zbuf, chunk, d, 0.0)
    base = sid * _RPT
    off = 0
    while off + chunk <= _RPT:
        pltpu.sync_copy(zbuf, acc_sh.at[pl.ds(base + off, chunk)])
        off += chunk
    if off < _RPT:
        rem = _RPT - off
        pltpu.sync_copy(zbuf.at[pl.ds(0, rem)], acc_sh.at[pl.ds(base + off, rem)])


def _make_deg_kernel(chunk):
    nch = _EPT // chunk

    @functools.partial(
        pl.kernel,
        out_type=jax.ShapeDtypeStruct((_NC, _N, _DEGK), jnp.float32),
        mesh=_mesh(),
        scratch_types=[
            pltpu.VMEM((_EPT,), jnp.int32),
            pltpu.VMEM((chunk, _DEGK), jnp.float32),
            pltpu.VMEM((chunk, _DEGK), jnp.float32),
            pltpu.VMEM_SHARED((_N, _DEGK), jnp.float32),
            pltpu.SemaphoreType.DMA,
        ],
        compiler_params=pltpu.CompilerParams(use_tc_tiling_on_sc=False),
    )
    def deg_kernel(edge_hbm, out_hbm, dst_v, ones_v, zbuf_v, acc_sh, sem_s):
        cid = lax.axis_index("c")
        sid = lax.axis_index("s")
        wid = cid * _NS + sid

        _zero_my_slice(acc_sh, zbuf_v, sid, _DEGK, chunk)
        _fill2d(ones_v, chunk, _DEGK, 1.0)
        pltpu.sync_copy(edge_hbm.at[1, pl.ds(wid * _EPT, _EPT)], dst_v)
        plsc.subcore_barrier()

        def s_start(k):
            pltpu.async_copy(
                ones_v, acc_sh.at[dst_v.at[pl.ds(k * chunk, chunk)]], sem_s, add=True
            )

        def s_wait():
            pltpu.make_async_copy(
                ones_v, acc_sh.at[dst_v.at[pl.ds(0, chunk)]], sem_s
            ).wait()

        s_start(0)
        s_start(1)

        def body(k, carry):
            s_wait()
            s_start(k + 2)
            return carry

        lax.fori_loop(0, nch - 2, body, 0)
        s_wait()
        s_wait()

        plsc.subcore_barrier()
        base = sid * _RPT
        pltpu.sync_copy(
            acc_sh.at[pl.ds(base, _RPT)], out_hbm.at[cid, pl.ds(base, _RPT)]
        )

    return deg_kernel


def _make_prop_kernel(d, chunk, gc, nb):
    nch = _EPT // chunk
    ng = nch // gc
    gce = gc * chunk

    @functools.partial(
        pl.kernel,
        out_type=jax.ShapeDtypeStruct((_NC, _N, d), jnp.float32),
        mesh=_mesh(),
        scratch_types=[
            pltpu.VMEM((gce,), jnp.int32),
            pltpu.VMEM((gce,), jnp.int32),
            pltpu.VMEM((nb, chunk, d), jnp.float32),
            pltpu.VMEM_SHARED((_N, d), jnp.float32),
            pltpu.SemaphoreType.DMA,
            pltpu.SemaphoreType.DMA,
        ],
        compiler_params=pltpu.CompilerParams(use_tc_tiling_on_sc=False),
    )
    def prop_kernel(g_hbm, edge_hbm, out_hbm, src_v, dst_v, rows_v, acc_sh, sem_g, sem_s):
        cid = lax.axis_index("c")
        sid = lax.axis_index("s")
        wid = cid * _NS + sid

        _zero_my_slice(acc_sh, rows_v.at[0], sid, d, chunk)
        plsc.subcore_barrier()

        def g_start(k, b):
            pltpu.make_async_copy(
                g_hbm.at[src_v.at[pl.ds(k * chunk, chunk)]], rows_v.at[b], sem_g
            ).start()

        def g_wait():
            pltpu.make_async_copy(
                g_hbm.at[src_v.at[pl.ds(0, chunk)]], rows_v.at[0], sem_g
            ).wait()

        def s_start(k, b):
            pltpu.async_copy(
                rows_v.at[b],
                acc_sh.at[dst_v.at[pl.ds(k * chunk, chunk)]],
                sem_s,
                add=True,
            )

        def s_wait():
            pltpu.make_async_copy(
                rows_v.at[0], acc_sh.at[dst_v.at[pl.ds(0, chunk)]], sem_s
            ).wait()

        def group_body(gi, carry):
            base = wid * _EPT + gi * gce
            pltpu.sync_copy(edge_hbm.at[0, pl.ds(base, gce)], src_v)
            pltpu.sync_copy(edge_hbm.at[1, pl.ds(base, gce)], dst_v)
            for j in range(nb - 1):
                g_start(j, j)

            def ring(kb, c2):
                for ph in range(nb):
                    k = nb * kb + ph
                    g_wait()
                    s_start(k, ph)
                    if ph == 0:
                        @pl.when(kb > 0)
                        def _():
                            s_wait()
                    else:
                        s_wait()

                    @pl.when(k < gc - (nb - 1))
                    def _():
                        g_start(k + nb - 1, (ph + nb - 1) % nb)

                return c2

            lax.fori_loop(0, gc // nb, ring, 0)
            s_wait()
            return carry

        lax.fori_loop(0, ng, group_body, 0)

        plsc.subcore_barrier()
        base = sid * _RPT
        pltpu.sync_copy(
            acc_sh.at[pl.ds(base, _RPT)], out_hbm.at[cid, pl.ds(base, _RPT)]
        )

    return prop_kernel


_deg_call = _make_deg_kernel(80)
_prop128 = _make_prop_kernel(128, 40, 250, 5)
_prop16 = _make_prop_kernel(16, 1000, 10, 5)

_ROWBLK = 2000
_NBLK = _N // _ROWBLK


def _dinv_block(dg_ref):
    degs = dg_ref[0] + dg_ref[1]
    deg = degs[:, 0:1]
    return lax.rsqrt(jnp.maximum(deg + 1.0, 1.0))


def _tc_mm_scale(x, w1, degp):
    def body(x_ref, w_ref, dg_ref, o_ref):
        p = jnp.dot(x_ref[...], w_ref[...], preferred_element_type=jnp.float32)
        o_ref[...] = p * _dinv_block(dg_ref)

    return pl.pallas_call(
        body,
        grid=(_NBLK,),
        in_specs=[
            pl.BlockSpec((_ROWBLK, 128), lambda i: (i, 0)),
            pl.BlockSpec((128, 128), lambda i: (0, 0)),
            pl.BlockSpec((_NC, _ROWBLK, _DEGK), lambda i: (0, i, 0)),
        ],
        out_specs=pl.BlockSpec((_ROWBLK, 128), lambda i: (i, 0)),
        out_shape=jax.ShapeDtypeStruct((_N, 128), jnp.float32),
    )(x, w1, degp)


def _tc_layer1(acc1, g1, degp, b1_2d, w2, w3):
    def body(a_ref, g_ref, dg_ref, b1_ref, w2_ref, w3_ref, o_ref):
        dinv = _dinv_block(dg_ref)
        h1 = jnp.maximum((a_ref[0] + a_ref[1] + g_ref[...]) * dinv + b1_ref[...], 0.0)
        w23 = jnp.dot(w2_ref[...], w3_ref[...], preferred_element_type=jnp.float32)
        o_ref[...] = jnp.dot(h1, w23, preferred_element_type=jnp.float32) * dinv

    return pl.pallas_call(
        body,
        grid=(_NBLK,),
        in_specs=[
            pl.BlockSpec((_NC, _ROWBLK, 128), lambda i: (0, i, 0)),
            pl.BlockSpec((_ROWBLK, 128), lambda i: (i, 0)),
            pl.BlockSpec((_NC, _ROWBLK, _DEGK), lambda i: (0, i, 0)),
            pl.BlockSpec((1, 128), lambda i: (0, 0)),
            pl.BlockSpec((128, 64), lambda i: (0, 0)),
            pl.BlockSpec((64, 16), lambda i: (0, 0)),
        ],
        out_specs=pl.BlockSpec((_ROWBLK, 16), lambda i: (i, 0)),
        out_shape=jax.ShapeDtypeStruct((_N, 16), jnp.float32),
    )(acc1, g1, degp, b1_2d, w2, w3)


def _tc_layer2(acc2, g2, degp, b2_2d, w3, b3_2d):
    def body(a_ref, g_ref, dg_ref, b2_ref, w3_ref, b3_ref, o_ref):
        dinv = _dinv_block(dg_ref)
        bias = jnp.dot(b2_ref[...], w3_ref[...], preferred_element_type=jnp.float32)
        z = (a_ref[0] + a_ref[1] + g_ref[...]) * dinv + bias + b3_ref[...]
        o_ref[...] = 1.0 / (1.0 + jnp.exp(-z))

    return pl.pallas_call(
        body,
        grid=(_NBLK,),
        in_specs=[
            pl.BlockSpec((_NC, _ROWBLK, 16), lambda i: (0, i, 0)),
            pl.BlockSpec((_ROWBLK, 16), lambda i: (i, 0)),
            pl.BlockSpec((_NC, _ROWBLK, _DEGK), lambda i: (0, i, 0)),
            pl.BlockSpec((1, 64), lambda i: (0, 0)),
            pl.BlockSpec((64, 16), lambda i: (0, 0)),
            pl.BlockSpec((1, 16), lambda i: (0, 0)),
        ],
        out_specs=pl.BlockSpec((_ROWBLK, 16), lambda i: (i, 0)),
        out_shape=jax.ShapeDtypeStruct((_N, 16), jnp.float32),
    )(acc2, g2, degp, b2_2d, w3, b3_2d)


def kernel(x, edge_index, W1, b1, W2, b2, W3, b3):
    b1_2d = b1.reshape(1, 128)
    b2_2d = b2.reshape(1, 64)
    b3_2d = b3.reshape(1, 16)

    degp = _deg_call(edge_index)
    g1 = _tc_mm_scale(x, W1, degp)
    acc1 = _prop128(g1, edge_index)
    g2 = _tc_layer1(acc1, g1, degp, b1_2d, W2, W3)
    acc2 = _prop16(g2, edge_index)
    return _tc_layer2(acc2, g2, degp, b2_2d, W3, b3_2d)


# speedup vs baseline: 1.0153x; 1.0153x over previous
import functools

import jax
import jax.numpy as jnp
from jax import lax
from jax.experimental import pallas as pl
from jax.experimental.pallas import tpu as pltpu
from jax.experimental.pallas import tpu_sc as plsc

_N = 10000
_E = 320000
_NC = 2
_NS = 16
_NW = _NC * _NS
_EPT = _E // _NW
_RPT = _N // _NS
_DEGK = 16


def _mesh():
    return plsc.VectorSubcoreMesh(
        core_axis_name="c", subcore_axis_name="s", num_cores=_NC, num_subcores=_NS
    )


def _fill2d(ref, rows, d, value):
    vec = jnp.full((16,), value, jnp.float32)
    nl = d // 16

    def body(i, carry):
        r = i // nl
        c = i - r * nl
        ref[r, pl.ds(c * 16, 16)] = vec
        return carry

    lax.fori_loop(0, rows * nl, body, 0)


def _zero_my_slice(acc_sh, zbuf, sid, d, chunk):
    _fill2d(zbuf, chunk, d, 0.0)
    base = sid * _RPT
    off = 0
    while off + chunk <= _RPT:
        pltpu.sync_copy(zbuf, acc_sh.at[pl.ds(base + off, chunk)])
        off += chunk
    if off < _RPT:
        rem = _RPT - off
        pltpu.sync_copy(zbuf.at[pl.ds(0, rem)], acc_sh.at[pl.ds(base + off, rem)])


def _make_deg_kernel(chunk):
    nch = _EPT // chunk

    @functools.partial(
        pl.kernel,
        out_type=jax.ShapeDtypeStruct((_NC, _N, _DEGK), jnp.float32),
        mesh=_mesh(),
        scratch_types=[
            pltpu.VMEM((_EPT,), jnp.int32),
            pltpu.VMEM((chunk, _DEGK), jnp.float32),
            pltpu.VMEM((chunk, _DEGK), jnp.float32),
            pltpu.VMEM_SHARED((_N, _DEGK), jnp.float32),
            pltpu.SemaphoreType.DMA,
        ],
        compiler_params=pltpu.CompilerParams(use_tc_tiling_on_sc=False),
    )
    def deg_kernel(edge_hbm, out_hbm, dst_v, ones_v, zbuf_v, acc_sh, sem_s):
        cid = lax.axis_index("c")
        sid = lax.axis_index("s")
        wid = cid * _NS + sid

        _zero_my_slice(acc_sh, zbuf_v, sid, _DEGK, chunk)
        _fill2d(ones_v, chunk, _DEGK, 1.0)
        pltpu.sync_copy(edge_hbm.at[1, pl.ds(wid * _EPT, _EPT)], dst_v)
        plsc.subcore_barrier()

        def s_start(k):
            pltpu.async_copy(
                ones_v, acc_sh.at[dst_v.at[pl.ds(k * chunk, chunk)]], sem_s, add=True
            )

        def s_wait():
            pltpu.make_async_copy(
                ones_v, acc_sh.at[dst_v.at[pl.ds(0, chunk)]], sem_s
            ).wait()

        s_start(0)
        s_start(1)

        def body(k, carry):
            s_wait()
            s_start(k + 2)
            return carry

        lax.fori_loop(0, nch - 2, body, 0)
        s_wait()
        s_wait()

        plsc.subcore_barrier()
        base = sid * _RPT
        pltpu.sync_copy(
            acc_sh.at[pl.ds(base, _RPT)], out_hbm.at[cid, pl.ds(base, _RPT)]
        )

    return deg_kernel


def _make_prop_kernel(d, chunk, gc, nb):
    nch = _EPT // chunk
    ng = nch // gc
    gce = gc * chunk

    @functools.partial(
        pl.kernel,
        out_type=jax.ShapeDtypeStruct((_NC, _N, d), jnp.float32),
        mesh=_mesh(),
        scratch_types=[
            pltpu.VMEM((gce,), jnp.int32),
            pltpu.VMEM((gce,), jnp.int32),
            pltpu.VMEM((nb, chunk, d), jnp.float32),
            pltpu.VMEM_SHARED((_N, d), jnp.float32),
            pltpu.SemaphoreType.DMA,
            pltpu.SemaphoreType.DMA,
        ],
        compiler_params=pltpu.CompilerParams(use_tc_tiling_on_sc=False),
    )
    def prop_kernel(g_hbm, edge_hbm, out_hbm, src_v, dst_v, rows_v, acc_sh, sem_g, sem_s):
        cid = lax.axis_index("c")
        sid = lax.axis_index("s")
        wid = cid * _NS + sid

        _zero_my_slice(acc_sh, rows_v.at[0], sid, d, chunk)
        plsc.subcore_barrier()

        def g_start(k, b):
            pltpu.make_async_copy(
                g_hbm.at[src_v.at[pl.ds(k * chunk, chunk)]], rows_v.at[b], sem_g
            ).start()

        def g_wait():
            pltpu.make_async_copy(
                g_hbm.at[src_v.at[pl.ds(0, chunk)]], rows_v.at[0], sem_g
            ).wait()

        def s_start(k, b):
            pltpu.async_copy(
                rows_v.at[b],
                acc_sh.at[dst_v.at[pl.ds(k * chunk, chunk)]],
                sem_s,
                add=True,
            )

        def s_wait():
            pltpu.make_async_copy(
                rows_v.at[0], acc_sh.at[dst_v.at[pl.ds(0, chunk)]], sem_s
            ).wait()

        def group_body(gi, carry):
            base = wid * _EPT + gi * gce
            pltpu.sync_copy(edge_hbm.at[0, pl.ds(base, gce)], src_v)
            pltpu.sync_copy(edge_hbm.at[1, pl.ds(base, gce)], dst_v)
            for j in range(nb - 1):
                g_start(j, j)

            def ring(kb, c2):
                for ph in range(nb):
                    k = nb * kb + ph
                    g_wait()
                    s_start(k, ph)
                    if ph == 0:
                        @pl.when(kb > 0)
                        def _():
                            s_wait()
                    else:
                        s_wait()

                    @pl.when(k < gc - (nb - 1))
                    def _():
                        g_start(k + nb - 1, (ph + nb - 1) % nb)

                return c2

            lax.fori_loop(0, gc // nb, ring, 0)
            s_wait()
            return carry

        lax.fori_loop(0, ng, group_body, 0)

        plsc.subcore_barrier()
        base = sid * _RPT
        pltpu.sync_copy(
            acc_sh.at[pl.ds(base, _RPT)], out_hbm.at[cid, pl.ds(base, _RPT)]
        )

    return prop_kernel


_deg_call = _make_deg_kernel(80)
_prop128 = _make_prop_kernel(128, 40, 250, 5)
_prop16 = _make_prop_kernel(16, 400, 25, 5)

_ROWBLK = 2000
_NBLK = _N // _ROWBLK


def _dinv_block(dg_ref):
    degs = dg_ref[0] + dg_ref[1]
    deg = degs[:, 0:1]
    return lax.rsqrt(jnp.maximum(deg + 1.0, 1.0))


def _tc_mm_scale(x, w1, degp):
    def body(x_ref, w_ref, dg_ref, o_ref):
        p = jnp.dot(x_ref[...], w_ref[...], preferred_element_type=jnp.float32)
        o_ref[...] = p * _dinv_block(dg_ref)

    return pl.pallas_call(
        body,
        grid=(_NBLK,),
        in_specs=[
            pl.BlockSpec((_ROWBLK, 128), lambda i: (i, 0)),
            pl.BlockSpec((128, 128), lambda i: (0, 0)),
            pl.BlockSpec((_NC, _ROWBLK, _DEGK), lambda i: (0, i, 0)),
        ],
        out_specs=pl.BlockSpec((_ROWBLK, 128), lambda i: (i, 0)),
        out_shape=jax.ShapeDtypeStruct((_N, 128), jnp.float32),
    )(x, w1, degp)


def _tc_layer1(acc1, g1, degp, b1_2d, w2, w3):
    def body(a_ref, g_ref, dg_ref, b1_ref, w2_ref, w3_ref, o_ref):
        dinv = _dinv_block(dg_ref)
        h1 = jnp.maximum((a_ref[0] + a_ref[1] + g_ref[...]) * dinv + b1_ref[...], 0.0)
        w23 = jnp.dot(w2_ref[...], w3_ref[...], preferred_element_type=jnp.float32)
        o_ref[...] = jnp.dot(h1, w23, preferred_element_type=jnp.float32) * dinv

    return pl.pallas_call(
        body,
        grid=(_NBLK,),
        in_specs=[
            pl.BlockSpec((_NC, _ROWBLK, 128), lambda i: (0, i, 0)),
            pl.BlockSpec((_ROWBLK, 128), lambda i: (i, 0)),
            pl.BlockSpec((_NC, _ROWBLK, _DEGK), lambda i: (0, i, 0)),
            pl.BlockSpec((1, 128), lambda i: (0, 0)),
            pl.BlockSpec((128, 64), lambda i: (0, 0)),
            pl.BlockSpec((64, 16), lambda i: (0, 0)),
        ],
        out_specs=pl.BlockSpec((_ROWBLK, 16), lambda i: (i, 0)),
        out_shape=jax.ShapeDtypeStruct((_N, 16), jnp.float32),
    )(acc1, g1, degp, b1_2d, w2, w3)


def _tc_layer2(acc2, g2, degp, b2_2d, w3, b3_2d):
    def body(a_ref, g_ref, dg_ref, b2_ref, w3_ref, b3_ref, o_ref):
        dinv = _dinv_block(dg_ref)
        bias = jnp.dot(b2_ref[...], w3_ref[...], preferred_element_type=jnp.float32)
        z = (a_ref[0] + a_ref[1] + g_ref[...]) * dinv + bias + b3_ref[...]
        o_ref[...] = 1.0 / (1.0 + jnp.exp(-z))

    return pl.pallas_call(
        body,
        grid=(_NBLK,),
        in_specs=[
            pl.BlockSpec((_NC, _ROWBLK, 16), lambda i: (0, i, 0)),
            pl.BlockSpec((_ROWBLK, 16), lambda i: (i, 0)),
            pl.BlockSpec((_NC, _ROWBLK, _DEGK), lambda i: (0, i, 0)),
            pl.BlockSpec((1, 64), lambda i: (0, 0)),
            pl.BlockSpec((64, 16), lambda i: (0, 0)),
            pl.BlockSpec((1, 16), lambda i: (0, 0)),
        ],
        out_specs=pl.BlockSpec((_ROWBLK, 16), lambda i: (i, 0)),
        out_shape=jax.ShapeDtypeStruct((_N, 16), jnp.float32),
    )(acc2, g2, degp, b2_2d, w3, b3_2d)


def kernel(x, edge_index, W1, b1, W2, b2, W3, b3):
    b1_2d = b1.reshape(1, 128)
    b2_2d = b2.reshape(1, 64)
    b3_2d = b3.reshape(1, 16)

    degp = _deg_call(edge_index)
    g1 = _tc_mm_scale(x, W1, degp)
    acc1 = _prop128(g1, edge_index)
    g2 = _tc_layer1(acc1, g1, degp, b1_2d, W2, W3)
    acc2 = _prop16(g2, edge_index)
    return _tc_layer2(acc2, g2, degp, b2_2d, W3, b3_2d)
